# Initial kernel scaffold; baseline (speedup 1.0000x reference)
import jax
import jax.numpy as jnp
from jax.experimental import pallas as pl


def kernel(values, indices):
    raise NotImplementedError("write your pallas kernel here")


# speedup vs baseline: 4.6871x; 4.6871x over previous
import functools

import jax
import jax.numpy as jnp
from jax import lax
from jax.experimental import pallas as pl
from jax.experimental.pallas import tpu as pltpu
from jax.experimental.pallas import tpu_sc as plsc

N = 4_000_000
OUT = 1_000_000
OUT_PAD = 1 << 20
NC, NS, L = 2, 16, 16
NW = NC * NS
RANGE = OUT_PAD // NW
CH = 8000
NCHUNK = N // CH


def _make_kernel():
  mesh = plsc.VectorSubcoreMesh(
      core_axis_name="c", subcore_axis_name="s", num_cores=NC, num_subcores=NS)

  @functools.partial(
      pl.kernel,
      out_type=jax.ShapeDtypeStruct((OUT_PAD,), jnp.float32),
      mesh=mesh,
      scratch_types=[
          pltpu.VMEM((CH,), jnp.int32),
          pltpu.VMEM((CH,), jnp.float32),
          pltpu.VMEM((RANGE,), jnp.float32),
      ],
      compiler_params=pltpu.CompilerParams(needs_layout_passes=False),
  )
  def scatter_kernel(idx_hbm, val_hbm, out_hbm, idx_v, val_v, acc_v):
    wid = lax.axis_index("c") * NS + lax.axis_index("s")
    lo = wid * RANGE

    zeros = jnp.zeros((L,), jnp.float32)

    @pl.loop(0, RANGE, step=L)
    def _init(i):
      acc_v[pl.ds(i, L)] = zeros

    @pl.loop(0, NCHUNK)
    def _chunk(k):
      base = k * CH
      pltpu.sync_copy(idx_hbm.at[pl.ds(base, CH)], idx_v)
      pltpu.sync_copy(val_hbm.at[pl.ds(base, CH)], val_v)

      @pl.loop(0, CH, step=L)
      def _vec(v):
        iv = idx_v[pl.ds(v, L)]
        xv = val_v[pl.ds(v, L)]
        mask = lax.shift_right_logical(iv, 15) == wid
        plsc.store_scatter(acc_v, [iv - lo], xv, mask=mask)

    pltpu.sync_copy(acc_v, out_hbm.at[pl.ds(lo, RANGE)])

  return scatter_kernel


_scatter = _make_kernel()


@jax.jit
def kernel(values, indices):
  out = _scatter(indices.astype(jnp.int32), values)
  return out[:OUT]


# speedup vs baseline: 13.8181x; 2.9481x over previous
import functools

import jax
import jax.numpy as jnp
from jax import lax
from jax.experimental import pallas as pl
from jax.experimental.pallas import tpu as pltpu
from jax.experimental.pallas import tpu_sc as plsc

N = 4_000_000
SEG = N // 2
OUT = 1_000_000
OUT_PAD = 1 << 20
NC, NS, L = 2, 16, 16
RSIZE = 65536
HALF = RSIZE // 2
CH = 8000
NCH = SEG // CH
PCH = 8192


def _make_kernel():
  mesh = plsc.VectorSubcoreMesh(
      core_axis_name="c", subcore_axis_name="s", num_cores=NC, num_subcores=NS)

  @functools.partial(
      pl.kernel,
      out_type=(jax.ShapeDtypeStruct((OUT_PAD,), jnp.float32),
                jax.ShapeDtypeStruct((NC * NS, HALF), jnp.int32)),
      mesh=mesh,
      scratch_types=[
          pltpu.VMEM((CH,), jnp.int32),
          pltpu.VMEM((CH,), jnp.int32),
          pltpu.VMEM((RSIZE,), jnp.int32),
          pltpu.VMEM((PCH,), jnp.int32),
          pltpu.VMEM((PCH,), jnp.int32),
          pltpu.VMEM((PCH,), jnp.int32),
          pltpu.VMEM((PCH,), jnp.float32),
          pltpu.VMEM((PCH,), jnp.float32),
          pltpu.SemaphoreType.DMA((2,)),
          pltpu.SemaphoreType.DMA,
      ],
      compiler_params=pltpu.CompilerParams(needs_layout_passes=False),
  )
  def scatter_kernel(idx_hbm, val_hbm, out_hbm, exch_hbm, idx_a, idx_b, pos,
                     pbuf, mbuf, gidx, gbuf, obuf, sems, gsem):
    idx_bufs = (idx_a, idx_b)
    c = lax.axis_index("c")
    s = lax.axis_index("s")
    wid = c * NS + s
    t = c * (NS // 2) + (s >> 1)
    h = s & 1
    lo = t * RSIZE
    seg_base = h * SEG

    iota = lax.iota(jnp.int32, L)
    neg1 = jnp.full((L,), -1, jnp.int32)

    @pl.loop(0, RSIZE, step=L)
    def _init(i):
      pos[pl.ds(i, L)] = neg1

    for b in range(2):
      pltpu.async_copy(
          idx_hbm.at[pl.ds(seg_base + b * CH, CH)], idx_bufs[b], sems.at[b])

    @pl.loop(0, NCH, step=2)
    def _chunk(k):
      for b in range(2):
        pltpu.make_async_copy(
            idx_hbm.at[pl.ds(0, CH)], idx_bufs[b], sems.at[b]).wait()

        jinit = seg_base + (k + b) * CH + iota

        @pl.loop(0, CH, step=L, init_carry=jinit, unroll=8)
        def _vec(i, jv):
          iv = idx_bufs[b][pl.ds(i, L)]
          mask = lax.shift_right_logical(iv, 16) == t
          local = lax.bitwise_and(iv, RSIZE - 1)
          plsc.store_scatter(pos, [local], jv, mask=mask)
          return jv + L

        @pl.when(k + b + 2 < NCH)
        def _issue():
          pltpu.async_copy(
              idx_hbm.at[pl.ds(seg_base + (k + b + 2) * CH, CH)],
              idx_bufs[b], sems.at[b])

    pltpu.sync_copy(pos.at[pl.ds((1 - h) * HALF, HALF)], exch_hbm.at[wid])
    plsc.subcore_barrier()
    pwid = c * NS + (s ^ 1)

    out_start = lo + h * HALF
    spread0 = wid * 100000

    @pl.loop(0, HALF, step=PCH)
    def _p2(cbase):
      pltpu.sync_copy(exch_hbm.at[pwid, pl.ds(cbase, PCH)], pbuf)

      @pl.loop(0, PCH, step=L)
      def _m1(i):
        own = pos[pl.ds(h * HALF + cbase + i, L)]
        par = pbuf[pl.ds(i, L)]
        m = lax.max(own, par)
        mbuf[pl.ds(i, L)] = m
        dummy = spread0 + cbase + i + iota
        gidx[pl.ds(i, L)] = jnp.where(m >= 0, m, dummy)

      pltpu.async_copy(val_hbm.at[gidx], gbuf, gsem).wait()

      @pl.loop(0, PCH, step=L)
      def _m2(i):
        m = mbuf[pl.ds(i, L)]
        g = gbuf[pl.ds(i, L)]
        obuf[pl.ds(i, L)] = jnp.where(m >= 0, g, 0.0)

      pltpu.sync_copy(obuf, out_hbm.at[pl.ds(out_start + cbase, PCH)])

  return scatter_kernel


_scatter = _make_kernel()


@jax.jit
def kernel(values, indices):
  out, _ = _scatter(indices.astype(jnp.int32), values)
  return out[:OUT]


# speedup vs baseline: 43.3469x; 3.1370x over previous
import functools

import jax
import jax.numpy as jnp
from jax import lax
from jax.experimental import pallas as pl
from jax.experimental.pallas import tpu as pltpu
from jax.experimental.pallas import tpu_sc as plsc

N = 4_000_000
SEG = N // 2
OUT = 1_000_000
OUT_PAD = 1 << 20
NC, NS, L = 2, 16, 16
NW = NC * NS
RSIZE = 65536
HALF = RSIZE // 2
CH = 8000
NCH = SEG // CH
W = 10
PCH = 4096
NPC = HALF // PCH


def _make_kernel():
  mesh = plsc.VectorSubcoreMesh(
      core_axis_name="c", subcore_axis_name="s", num_cores=NC, num_subcores=NS)

  @functools.partial(
      pl.kernel,
      out_type=(jax.ShapeDtypeStruct((OUT_PAD,), jnp.float32),
                jax.ShapeDtypeStruct((NW, HALF), jnp.int32)),
      mesh=mesh,
      scratch_types=[
          pltpu.VMEM((CH,), jnp.int32),
          pltpu.VMEM((CH,), jnp.int32),
          pltpu.VMEM((RSIZE,), jnp.int32),
          pltpu.VMEM((PCH,), jnp.int32),
          pltpu.VMEM((PCH,), jnp.int32),
          pltpu.VMEM((PCH,), jnp.int32),
          pltpu.VMEM((PCH,), jnp.int32),
          pltpu.VMEM((PCH,), jnp.int32),
          pltpu.VMEM((PCH,), jnp.int32),
          pltpu.VMEM((PCH,), jnp.float32),
          pltpu.VMEM((PCH,), jnp.float32),
          pltpu.VMEM((PCH,), jnp.float32),
          pltpu.SemaphoreType.DMA((2,)),
          pltpu.SemaphoreType.DMA((2,)),
          pltpu.SemaphoreType.DMA((2,)),
      ],
      compiler_params=pltpu.CompilerParams(needs_layout_passes=False),
  )
  def scatter_kernel(idx_hbm, val_hbm, out_hbm, exch_hbm, idx_a, idx_b, pos,
                     pb_a, pb_b, mb_a, mb_b, gi_a, gi_b, gv_a, gv_b, obuf,
                     sems, psems, gsems):
    idx_bufs = (idx_a, idx_b)
    pbufs = (pb_a, pb_b)
    mbufs = (mb_a, mb_b)
    gidxs = (gi_a, gi_b)
    gbufs = (gv_a, gv_b)

    c = lax.axis_index("c")
    s = lax.axis_index("s")
    wid = c * NS + s
    t = c * (NS // 2) + (s >> 1)
    h = s & 1
    lo = t * RSIZE
    seg_base = h * SEG

    iota = lax.iota(jnp.int32, L)
    neg1 = jnp.full((L,), -1, jnp.int32)

    @pl.loop(0, RSIZE, step=L)
    def _init(i):
      pos[pl.ds(i, L)] = neg1

    for b in range(2):
      pltpu.async_copy(
          idx_hbm.at[pl.ds(seg_base + b * CH, CH)], idx_bufs[b], sems.at[b])

    @pl.loop(0, NCH, step=2)
    def _chunk(k):
      for b in range(2):
        pltpu.make_async_copy(
            idx_hbm.at[pl.ds(0, CH)], idx_bufs[b], sems.at[b]).wait()

        cbase = seg_base + (k + b) * CH

        @pl.loop(0, CH, step=W * L)
        def _win(i):
          ivs = [idx_bufs[b][pl.ds(i + kk * L, L)] for kk in range(W)]
          jw = (cbase + i) + iota
          for kk in range(W):
            local = ivs[kk] - lo
            mask = plsc.bitcast(local, jnp.uint32) < jnp.uint32(RSIZE)
            plsc.store_scatter(pos, [local], jw + kk * L, mask=mask)

        @pl.when(k + b + 2 < NCH)
        def _issue():
          pltpu.async_copy(
              idx_hbm.at[pl.ds(seg_base + (k + b + 2) * CH, CH)],
              idx_bufs[b], sems.at[b])

    pltpu.sync_copy(pos.at[pl.ds((1 - h) * HALF, HALF)], exch_hbm.at[wid])
    plsc.subcore_barrier()
    pwid = c * NS + (s ^ 1)

    out_start = lo + h * HALF
    spread0 = wid * 100000

    def fetch_partner(cc):
      pltpu.async_copy(
          exch_hbm.at[pwid, pl.ds(cc * PCH, PCH)], pbufs[cc % 2],
          psems.at[cc % 2])

    def wait_partner(cc):
      pltpu.make_async_copy(
          exch_hbm.at[pwid, pl.ds(0, PCH)], pbufs[cc % 2],
          psems.at[cc % 2]).wait()

    def combine_and_start_gather(cc):
      p = cc % 2
      pb, mb, gi = pbufs[p], mbufs[p], gidxs[p]
      cbase = cc * PCH

      @pl.loop(0, PCH, step=L)
      def _m1(i):
        own = pos[pl.ds(h * HALF + cbase + i, L)]
        par = pb[pl.ds(i, L)]
        m = lax.max(own, par)
        mb[pl.ds(i, L)] = m
        dummy = spread0 + cbase + i + iota
        gi[pl.ds(i, L)] = jnp.where(m >= 0, m, dummy)

      pltpu.async_copy(val_hbm.at[gi], gbufs[p], gsems.at[p])

    def finish_chunk(cc):
      p = cc % 2
      pltpu.make_async_copy(
          val_hbm.at[pl.ds(0, PCH)], gbufs[p], gsems.at[p]).wait()
      mb, gv = mbufs[p], gbufs[p]

      @pl.loop(0, PCH, step=L)
      def _m2(i):
        m = mb[pl.ds(i, L)]
        g = gv[pl.ds(i, L)]
        obuf[pl.ds(i, L)] = jnp.where(m >= 0, g, 0.0)

      pltpu.sync_copy(obuf, out_hbm.at[pl.ds(out_start + cc * PCH, PCH)])

    fetch_partner(0)
    for cc in range(NPC):
      wait_partner(cc)
      if cc + 1 < NPC:
        fetch_partner(cc + 1)
      combine_and_start_gather(cc)
      if cc > 0:
        finish_chunk(cc - 1)
    finish_chunk(NPC - 1)

  return scatter_kernel


_scatter = _make_kernel()


@jax.jit
def kernel(values, indices):
  out, _ = _scatter(indices.astype(jnp.int32), values)
  return out[:OUT]
